# baseline (speedup 1.0000x reference)
import functools

import jax
import jax.numpy as jnp
from jax import lax
from jax.experimental import pallas as pl
from jax.experimental.pallas import tpu as pltpu
from jax.experimental.pallas import tpu_sc as plsc

BN_EPS = 1e-5
LANES = 128
EBLK = 128
NSUB = 16
NCORES = 2
RB = 400


def _round_up(v, m):
    return (v + m - 1) // m * m


NQ = 2


@functools.partial(jax.jit, static_argnames=("F", "nb", "npad"))
def _sc_segment_sum(h_ch, src2d, dst2d, zeros_pad, F, nb, npad):
    F_per_core = F // NCORES
    ZR = npad // NSUB
    qb = nb // NQ
    mesh = plsc.VectorSubcoreMesh(core_axis_name="c", subcore_axis_name="s")

    @functools.partial(
        pl.kernel,
        out_type=jax.ShapeDtypeStruct((F, npad, LANES), jnp.float32),
        mesh=mesh,
        scratch_types=[
            pltpu.VMEM((qb, EBLK), jnp.int32),
            pltpu.VMEM((qb, EBLK), jnp.int32),
            pltpu.VMEM((EBLK, LANES), jnp.float32),
            pltpu.VMEM((EBLK, LANES), jnp.float32),
            pltpu.VMEM_SHARED((npad, LANES), jnp.float32),
            pltpu.SemaphoreType.DMA,
            pltpu.SemaphoreType.DMA,
        ],
    )
    def segsum(h_hbm, src_hbm, dst_hbm, z_hbm, out_hbm,
               src_v, dst_v, buf0, buf1, acc, sem0, sem1):
        c = lax.axis_index("c")
        s = lax.axis_index("s")
        bufs = (buf0, buf1)
        sems = (sem0, sem1)
        for fi in range(F_per_core):
            chunk = c * F_per_core + fi
            h_f = h_hbm.at[chunk]
            pltpu.sync_copy(z_hbm.at[pl.ds(s * ZR, ZR)],
                            acc.at[pl.ds(s * ZR, ZR)])
            plsc.subcore_barrier()

            def start(j, k, h_f=h_f):
                pltpu.make_async_copy(h_f.at[src_v.at[j]], bufs[k],
                                      sems[k]).start()

            def wait(k, h_f=h_f):
                pltpu.make_async_copy(h_f.at[src_v.at[0]], bufs[k],
                                      sems[k]).wait()

            for q in range(NQ):
                pltpu.sync_copy(src_hbm.at[pl.ds((s * NQ + q) * qb, qb)],
                                src_v)
                pltpu.sync_copy(dst_hbm.at[pl.ds((s * NQ + q) * qb, qb)],
                                dst_v)
                start(0, 0)

                def body(i, carry):
                    j0 = 2 * i
                    start(j0 + 1, 1)
                    wait(0)
                    pltpu.sync_copy(bufs[0], acc.at[dst_v.at[j0]], add=True)

                    @pl.when(j0 + 2 < qb)
                    def _():
                        start(j0 + 2, 0)

                    wait(1)
                    pltpu.sync_copy(bufs[1], acc.at[dst_v.at[j0 + 1]],
                                    add=True)
                    return carry

                lax.fori_loop(0, qb // 2, body, 0)
            plsc.subcore_barrier()
            pltpu.sync_copy(acc.at[pl.ds(s * ZR, ZR)],
                            out_hbm.at[chunk, pl.ds(s * ZR, ZR)])
            if fi + 1 < F_per_core:
                plsc.subcore_barrier()

    return segsum(h_ch, src2d, dst2d, zeros_pad)


def _mlp_body(eps_ref, h_ref, agg_ref, w1_ref, b1_ref, w2_ref, b2_ref,
              z2_ref, s1_ref, s2_ref):
    i = pl.program_id(0)
    F = h_ref.shape[0]
    e = eps_ref[0, 0]
    acc = jnp.zeros((RB, w1_ref.shape[2]), jnp.float32)
    for f in range(F):
        zf = e * h_ref[f] + agg_ref[f]
        acc += jnp.dot(zf.astype(jnp.bfloat16),
                       w1_ref[f].astype(jnp.bfloat16),
                       preferred_element_type=jnp.float32)
    a = jnp.maximum(acc + b1_ref[...], 0.0)
    z2 = jnp.dot(a.astype(jnp.bfloat16), w2_ref[...].astype(jnp.bfloat16),
                 preferred_element_type=jnp.float32) + b2_ref[...]
    z2_ref[...] = z2
    ps1 = jnp.sum(z2, axis=0, keepdims=True)
    ps2 = jnp.sum(z2 * z2, axis=0, keepdims=True)

    @pl.when(i == 0)
    def _():
        s1_ref[...] = ps1
        s2_ref[...] = ps2

    @pl.when(i > 0)
    def _():
        s1_ref[...] += ps1
        s2_ref[...] += ps2


def _tc_mlp(eps1p, h_ch, agg_ch, w1r, b1, w2, b2, n_rows):
    F = h_ch.shape[0]
    hid = w2.shape[1]
    grid = n_rows // RB
    return pl.pallas_call(
        _mlp_body,
        grid=(grid,),
        in_specs=[
            pl.BlockSpec((1, 1), lambda i: (0, 0), memory_space=pltpu.SMEM),
            pl.BlockSpec((F, RB, LANES), lambda i: (0, i, 0)),
            pl.BlockSpec((F, RB, LANES), lambda i: (0, i, 0)),
            pl.BlockSpec((F, LANES, hid), lambda i: (0, 0, 0)),
            pl.BlockSpec((1, hid), lambda i: (0, 0)),
            pl.BlockSpec((hid, hid), lambda i: (0, 0)),
            pl.BlockSpec((1, hid), lambda i: (0, 0)),
        ],
        out_specs=[
            pl.BlockSpec((RB, hid), lambda i: (i, 0)),
            pl.BlockSpec((1, hid), lambda i: (0, 0)),
            pl.BlockSpec((1, hid), lambda i: (0, 0)),
        ],
        out_shape=[
            jax.ShapeDtypeStruct((n_rows, hid), jnp.float32),
            jax.ShapeDtypeStruct((1, hid), jnp.float32),
            jax.ShapeDtypeStruct((1, hid), jnp.float32),
        ],
    )(eps1p, h_ch, agg_ch, w1r, b1, w2, b2)


def _bn_body(z2_ref, s1_ref, s2_ref, gamma_ref, beta_ref, n_ref, out_ref):
    inv_n = 1.0 / n_ref[0, 0]
    mean = s1_ref[...] * inv_n
    var = s2_ref[...] * inv_n - mean * mean
    a = gamma_ref[...] * lax.rsqrt(var + BN_EPS)
    b = beta_ref[...] - mean * a
    h = jnp.maximum(z2_ref[...] * a + b, 0.0)
    F = out_ref.shape[0]
    for f in range(F):
        out_ref[f] = h[:, f * LANES:(f + 1) * LANES]


def _tc_bn(z2, s1, s2, gamma, beta, nf, n_rows):
    hid = z2.shape[1]
    F = hid // LANES
    grid = n_rows // RB
    return pl.pallas_call(
        _bn_body,
        grid=(grid,),
        in_specs=[
            pl.BlockSpec((RB, hid), lambda i: (i, 0)),
            pl.BlockSpec((1, hid), lambda i: (0, 0)),
            pl.BlockSpec((1, hid), lambda i: (0, 0)),
            pl.BlockSpec((1, hid), lambda i: (0, 0)),
            pl.BlockSpec((1, hid), lambda i: (0, 0)),
            pl.BlockSpec((1, 1), lambda i: (0, 0), memory_space=pltpu.SMEM),
        ],
        out_specs=pl.BlockSpec((F, RB, LANES), lambda i: (0, i, 0)),
        out_shape=jax.ShapeDtypeStruct((F, n_rows, LANES), jnp.float32),
    )(z2, s1, s2, gamma, beta, nf)


def _pool_body(bat_ref, h_ref, wc_ref, bc_ref, out_ref, acc_ref):
    i = pl.program_id(0)
    ng = acc_ref.shape[0]
    F = h_ref.shape[0]
    bat = bat_ref[0, 0]
    ohT = (lax.broadcasted_iota(jnp.int32, (ng, RB), 0)
           == bat[None, :]).astype(jnp.float32)
    for f in range(F):
        part = jnp.dot(ohT, h_ref[f], preferred_element_type=jnp.float32)

        @pl.when(i == 0)
        def _():
            acc_ref[:, f * LANES:(f + 1) * LANES] = part

        @pl.when(i > 0)
        def _():
            acc_ref[:, f * LANES:(f + 1) * LANES] += part

    @pl.when(i == pl.num_programs(0) - 1)
    def _():
        out_ref[...] = jnp.dot(acc_ref[...], wc_ref[...],
                               preferred_element_type=jnp.float32) \
            + bc_ref[...]


def _tc_pool(bat3d, h_ch, wc, bc, n_rows, num_graphs):
    F = h_ch.shape[0]
    hid = F * LANES
    ncls = wc.shape[1]
    grid = n_rows // RB
    return pl.pallas_call(
        _pool_body,
        grid=(grid,),
        in_specs=[
            pl.BlockSpec((1, 1, RB), lambda i: (i, 0, 0)),
            pl.BlockSpec((F, RB, LANES), lambda i: (0, i, 0)),
            pl.BlockSpec((hid, ncls), lambda i: (0, 0)),
            pl.BlockSpec((1, ncls), lambda i: (0, 0)),
        ],
        out_specs=pl.BlockSpec((num_graphs, ncls), lambda i: (0, 0)),
        out_shape=jax.ShapeDtypeStruct((num_graphs, ncls), jnp.float32),
        scratch_shapes=[pltpu.VMEM((num_graphs, hid), jnp.float32)],
    )(bat3d, h_ch, wc, bc)


def kernel(x, edge_index, batch, params):
    n, in_dim = x.shape
    e_edges = edge_index.shape[1]
    num_graphs = 64
    ncls = params['Wc'].shape[1]

    npad = _round_up(n + NSUB, NSUB * 8)
    nb = _round_up((e_edges + NSUB * EBLK - 1) // (NSUB * EBLK), 8 * NQ)
    e_pad = NSUB * nb * EBLK - e_edges

    src = edge_index[0]
    dst = edge_index[1]
    pad_ar = jnp.arange(e_pad, dtype=jnp.int32)
    src_p = jnp.concatenate([src, (pad_ar * 1009) % n]).reshape(NSUB * nb, EBLK)
    dst_p = jnp.concatenate([dst, n + (pad_ar % NSUB)]).reshape(NSUB * nb, EBLK)
    zeros_pad = jnp.zeros((npad, LANES), jnp.float32)

    h_ch = x.reshape(n, in_dim // LANES, LANES).transpose(1, 0, 2)

    nf = jnp.full((1, 1), float(n), jnp.float32)
    for lp in params['layers']:
        F = h_ch.shape[0]
        agg_ch = _sc_segment_sum(h_ch, src_p, dst_p, zeros_pad,
                                 F=F, nb=nb, npad=npad)
        eps1p = (1.0 + lp['eps']).reshape(1, 1).astype(jnp.float32)
        hid = lp['W2'].shape[1]
        w1r = lp['W1'].reshape(F, LANES, hid)
        z2, s1, s2 = _tc_mlp(eps1p, h_ch, agg_ch, w1r,
                             lp['b1'].reshape(1, hid), lp['W2'],
                             lp['b2'].reshape(1, hid), n)
        h_ch = _tc_bn(z2, s1, s2, lp['gamma'].reshape(1, hid),
                      lp['beta'].reshape(1, hid), nf, n)

    bat3d = batch.reshape(n // RB, 1, RB)
    return _tc_pool(bat3d, h_ch, params['Wc'],
                    params['bc'].reshape(1, ncls), n, num_graphs)


# speedup vs baseline: 1.0851x; 1.0851x over previous
import functools

import jax
import jax.numpy as jnp
from jax import lax
from jax.experimental import pallas as pl
from jax.experimental.pallas import tpu as pltpu
from jax.experimental.pallas import tpu_sc as plsc

BN_EPS = 1e-5
LANES = 128
EBLK = 128
NSUB = 16
NCORES = 2
RB = 400


def _round_up(v, m):
    return (v + m - 1) // m * m


NQ = 2


@functools.partial(jax.jit, static_argnames=("F", "nb", "npad"))
def _sc_segment_sum(h_ch, src2d, dst2d, zeros_pad, F, nb, npad):
    F_per_core = F // NCORES
    ZR = npad // NSUB
    qb = nb // NQ
    mesh = plsc.VectorSubcoreMesh(core_axis_name="c", subcore_axis_name="s")

    @functools.partial(
        pl.kernel,
        out_type=jax.ShapeDtypeStruct((F, npad, LANES), jnp.float32),
        mesh=mesh,
        scratch_types=[
            pltpu.VMEM((qb, EBLK), jnp.int32),
            pltpu.VMEM((qb, EBLK), jnp.int32),
            pltpu.VMEM((EBLK, LANES), jnp.float32),
            pltpu.VMEM((EBLK, LANES), jnp.float32),
            pltpu.VMEM_SHARED((npad, LANES), jnp.float32),
            pltpu.SemaphoreType.DMA,
            pltpu.SemaphoreType.DMA,
        ],
    )
    def segsum(h_hbm, src_hbm, dst_hbm, z_hbm, out_hbm,
               src_v, dst_v, buf0, buf1, acc, sem0, sem1):
        c = lax.axis_index("c")
        s = lax.axis_index("s")
        bufs = (buf0, buf1)
        sems = (sem0, sem1)
        for fi in range(F_per_core):
            chunk = c * F_per_core + fi
            h_f = h_hbm.at[chunk]
            pltpu.sync_copy(z_hbm.at[pl.ds(s * ZR, ZR)],
                            acc.at[pl.ds(s * ZR, ZR)])
            plsc.subcore_barrier()

            def start(j, k, h_f=h_f):
                pltpu.make_async_copy(h_f.at[src_v.at[j]], bufs[k],
                                      sems[k]).start()

            def wait(k, h_f=h_f):
                pltpu.make_async_copy(h_f.at[src_v.at[0]], bufs[k],
                                      sems[k]).wait()

            for q in range(NQ):
                pltpu.sync_copy(src_hbm.at[pl.ds((s * NQ + q) * qb, qb)],
                                src_v)
                pltpu.sync_copy(dst_hbm.at[pl.ds((s * NQ + q) * qb, qb)],
                                dst_v)
                start(0, 0)

                def body(i, carry):
                    j0 = 2 * i
                    start(j0 + 1, 1)
                    wait(0)
                    pltpu.sync_copy(bufs[0], acc.at[dst_v.at[j0]], add=True)

                    @pl.when(j0 + 2 < qb)
                    def _():
                        start(j0 + 2, 0)

                    wait(1)
                    pltpu.sync_copy(bufs[1], acc.at[dst_v.at[j0 + 1]],
                                    add=True)
                    return carry

                lax.fori_loop(0, qb // 2, body, 0)
            plsc.subcore_barrier()
            pltpu.sync_copy(acc.at[pl.ds(s * ZR, ZR)],
                            out_hbm.at[chunk, pl.ds(s * ZR, ZR)])
            if fi + 1 < F_per_core:
                plsc.subcore_barrier()

    return segsum(h_ch, src2d, dst2d, zeros_pad)


def _mlp_bn_body(eps_ref, h_ref, agg_ref, w1_ref, b1_ref, w2_ref, b2_ref,
                 gamma_ref, beta_ref, n_ref, out_ref, z2s_ref, s1_ref,
                 s2_ref):
    p = pl.program_id(0)
    i = pl.program_id(1)
    F = h_ref.shape[0]

    @pl.when(p == 0)
    def _():
        e = eps_ref[0, 0]
        acc = jnp.zeros((RB, w1_ref.shape[2]), jnp.float32)
        for f in range(F):
            zf = e * h_ref[f] + agg_ref[f]
            acc += jnp.dot(zf.astype(jnp.bfloat16),
                           w1_ref[f].astype(jnp.bfloat16),
                           preferred_element_type=jnp.float32)
        a = jnp.maximum(acc + b1_ref[...], 0.0)
        z2 = jnp.dot(a.astype(jnp.bfloat16), w2_ref[...].astype(jnp.bfloat16),
                     preferred_element_type=jnp.float32) + b2_ref[...]
        z2s_ref[pl.ds(i * RB, RB), :] = z2
        ps1 = jnp.sum(z2, axis=0, keepdims=True)
        ps2 = jnp.sum(z2 * z2, axis=0, keepdims=True)

        @pl.when(i == 0)
        def _():
            s1_ref[...] = ps1
            s2_ref[...] = ps2

        @pl.when(i > 0)
        def _():
            s1_ref[...] += ps1
            s2_ref[...] += ps2

    @pl.when(p == 1)
    def _():
        inv_n = 1.0 / n_ref[0, 0]
        mean = s1_ref[...] * inv_n
        var = s2_ref[...] * inv_n - mean * mean
        a = gamma_ref[...] * lax.rsqrt(var + BN_EPS)
        b = beta_ref[...] - mean * a
        h = jnp.maximum(z2s_ref[pl.ds(i * RB, RB), :] * a + b, 0.0)
        Fo = out_ref.shape[0]
        for f in range(Fo):
            out_ref[f] = h[:, f * LANES:(f + 1) * LANES]


def _tc_mlp_bn(eps1p, h_ch, agg_ch, w1r, b1, w2, b2, gamma, beta, nf,
               n_rows):
    F = h_ch.shape[0]
    hid = w2.shape[1]
    Fo = hid // LANES
    grid = n_rows // RB
    return pl.pallas_call(
        _mlp_bn_body,
        grid=(2, grid),
        in_specs=[
            pl.BlockSpec((1, 1), lambda p, i: (0, 0),
                         memory_space=pltpu.SMEM),
            pl.BlockSpec((F, RB, LANES), lambda p, i: (0, i * (1 - p), 0)),
            pl.BlockSpec((F, RB, LANES), lambda p, i: (0, i * (1 - p), 0)),
            pl.BlockSpec((F, LANES, hid), lambda p, i: (0, 0, 0)),
            pl.BlockSpec((1, hid), lambda p, i: (0, 0)),
            pl.BlockSpec((hid, hid), lambda p, i: (0, 0)),
            pl.BlockSpec((1, hid), lambda p, i: (0, 0)),
            pl.BlockSpec((1, hid), lambda p, i: (0, 0)),
            pl.BlockSpec((1, hid), lambda p, i: (0, 0)),
            pl.BlockSpec((1, 1), lambda p, i: (0, 0),
                         memory_space=pltpu.SMEM),
        ],
        out_specs=pl.BlockSpec((Fo, RB, LANES), lambda p, i: (0, i * p, 0)),
        out_shape=jax.ShapeDtypeStruct((Fo, n_rows, LANES), jnp.float32),
        scratch_shapes=[
            pltpu.VMEM((n_rows, hid), jnp.float32),
            pltpu.VMEM((1, hid), jnp.float32),
            pltpu.VMEM((1, hid), jnp.float32),
        ],
        compiler_params=pltpu.CompilerParams(
            dimension_semantics=("arbitrary", "arbitrary")),
    )(eps1p, h_ch, agg_ch, w1r, b1, w2, b2, gamma, beta, nf)


def _mlp_bn_pool_body(eps_ref, h_ref, agg_ref, w1_ref, b1_ref, w2_ref,
                      b2_ref, gamma_ref, beta_ref, n_ref, bat_ref, wc_ref,
                      bc_ref, out_ref, z2s_ref, s1_ref, s2_ref, pacc_ref):
    p = pl.program_id(0)
    i = pl.program_id(1)
    ngrid = pl.num_programs(1)
    F = h_ref.shape[0]
    ng = pacc_ref.shape[0]

    @pl.when(p == 0)
    def _():
        e = eps_ref[0, 0]
        acc = jnp.zeros((RB, w1_ref.shape[2]), jnp.float32)
        for f in range(F):
            zf = e * h_ref[f] + agg_ref[f]
            acc += jnp.dot(zf.astype(jnp.bfloat16),
                           w1_ref[f].astype(jnp.bfloat16),
                           preferred_element_type=jnp.float32)
        a = jnp.maximum(acc + b1_ref[...], 0.0)
        z2 = jnp.dot(a.astype(jnp.bfloat16), w2_ref[...].astype(jnp.bfloat16),
                     preferred_element_type=jnp.float32) + b2_ref[...]
        z2s_ref[pl.ds(i * RB, RB), :] = z2
        ps1 = jnp.sum(z2, axis=0, keepdims=True)
        ps2 = jnp.sum(z2 * z2, axis=0, keepdims=True)

        @pl.when(i == 0)
        def _():
            s1_ref[...] = ps1
            s2_ref[...] = ps2

        @pl.when(i > 0)
        def _():
            s1_ref[...] += ps1
            s2_ref[...] += ps2

    @pl.when(p == 1)
    def _():
        inv_n = 1.0 / n_ref[0, 0]
        mean = s1_ref[...] * inv_n
        var = s2_ref[...] * inv_n - mean * mean
        a = gamma_ref[...] * lax.rsqrt(var + BN_EPS)
        b = beta_ref[...] - mean * a
        h = jnp.maximum(z2s_ref[pl.ds(i * RB, RB), :] * a + b, 0.0)
        bat = bat_ref[0, 0]
        ohT = (lax.broadcasted_iota(jnp.int32, (ng, RB), 0)
               == bat[None, :]).astype(jnp.float32)
        part = jnp.dot(ohT, h, preferred_element_type=jnp.float32)

        @pl.when(i == 0)
        def _():
            pacc_ref[...] = part

        @pl.when(i > 0)
        def _():
            pacc_ref[...] += part

        @pl.when(i == ngrid - 1)
        def _():
            out_ref[...] = jnp.dot(pacc_ref[...], wc_ref[...],
                                   preferred_element_type=jnp.float32) \
                + bc_ref[...]


def _tc_mlp_bn_pool(eps1p, h_ch, agg_ch, w1r, b1, w2, b2, gamma, beta, nf,
                    bat3d, wc, bc, n_rows, num_graphs):
    F = h_ch.shape[0]
    hid = w2.shape[1]
    ncls = wc.shape[1]
    grid = n_rows // RB
    return pl.pallas_call(
        _mlp_bn_pool_body,
        grid=(2, grid),
        in_specs=[
            pl.BlockSpec((1, 1), lambda p, i: (0, 0),
                         memory_space=pltpu.SMEM),
            pl.BlockSpec((F, RB, LANES), lambda p, i: (0, i * (1 - p), 0)),
            pl.BlockSpec((F, RB, LANES), lambda p, i: (0, i * (1 - p), 0)),
            pl.BlockSpec((F, LANES, hid), lambda p, i: (0, 0, 0)),
            pl.BlockSpec((1, hid), lambda p, i: (0, 0)),
            pl.BlockSpec((hid, hid), lambda p, i: (0, 0)),
            pl.BlockSpec((1, hid), lambda p, i: (0, 0)),
            pl.BlockSpec((1, hid), lambda p, i: (0, 0)),
            pl.BlockSpec((1, hid), lambda p, i: (0, 0)),
            pl.BlockSpec((1, 1), lambda p, i: (0, 0),
                         memory_space=pltpu.SMEM),
            pl.BlockSpec((1, 1, RB), lambda p, i: (i, 0, 0)),
            pl.BlockSpec((hid, ncls), lambda p, i: (0, 0)),
            pl.BlockSpec((1, ncls), lambda p, i: (0, 0)),
        ],
        out_specs=pl.BlockSpec((num_graphs, ncls), lambda p, i: (0, 0)),
        out_shape=jax.ShapeDtypeStruct((num_graphs, ncls), jnp.float32),
        scratch_shapes=[
            pltpu.VMEM((n_rows, hid), jnp.float32),
            pltpu.VMEM((1, hid), jnp.float32),
            pltpu.VMEM((1, hid), jnp.float32),
            pltpu.VMEM((num_graphs, hid), jnp.float32),
        ],
        compiler_params=pltpu.CompilerParams(
            dimension_semantics=("arbitrary", "arbitrary")),
    )(eps1p, h_ch, agg_ch, w1r, b1, w2, b2, gamma, beta, nf, bat3d, wc, bc)


def kernel(x, edge_index, batch, params):
    n, in_dim = x.shape
    e_edges = edge_index.shape[1]
    num_graphs = 64
    ncls = params['Wc'].shape[1]

    npad = _round_up(n + NSUB, NSUB * 8)
    nb = _round_up((e_edges + NSUB * EBLK - 1) // (NSUB * EBLK), 8 * NQ)
    e_pad = NSUB * nb * EBLK - e_edges

    src = edge_index[0]
    dst = edge_index[1]
    pad_ar = jnp.arange(e_pad, dtype=jnp.int32)
    src_p = jnp.concatenate([src, (pad_ar * 1009) % n]).reshape(NSUB * nb, EBLK)
    dst_p = jnp.concatenate([dst, n + (pad_ar % NSUB)]).reshape(NSUB * nb, EBLK)
    zeros_pad = jnp.zeros((npad, LANES), jnp.float32)

    h_ch = x.reshape(n, in_dim // LANES, LANES).transpose(1, 0, 2)

    nf = jnp.full((1, 1), float(n), jnp.float32)
    bat3d = batch.reshape(n // RB, 1, RB)
    n_layers = len(params['layers'])
    for li, lp in enumerate(params['layers']):
        F = h_ch.shape[0]
        agg_ch = _sc_segment_sum(h_ch, src_p, dst_p, zeros_pad,
                                 F=F, nb=nb, npad=npad)
        eps1p = (1.0 + lp['eps']).reshape(1, 1).astype(jnp.float32)
        hid = lp['W2'].shape[1]
        w1r = lp['W1'].reshape(F, LANES, hid)
        args = (eps1p, h_ch, agg_ch, w1r, lp['b1'].reshape(1, hid),
                lp['W2'], lp['b2'].reshape(1, hid),
                lp['gamma'].reshape(1, hid), lp['beta'].reshape(1, hid), nf)
        if li + 1 < n_layers:
            h_ch = _tc_mlp_bn(*args, n)
        else:
            return _tc_mlp_bn_pool(*args, bat3d, params['Wc'],
                                   params['bc'].reshape(1, ncls), n,
                                   num_graphs)


# speedup vs baseline: 1.0945x; 1.0087x over previous
import functools

import jax
import jax.numpy as jnp
from jax import lax
from jax.experimental import pallas as pl
from jax.experimental.pallas import tpu as pltpu
from jax.experimental.pallas import tpu_sc as plsc

BN_EPS = 1e-5
LANES = 128
EBLK = 128
NSUB = 16
NCORES = 2
RB = 400


def _round_up(v, m):
    return (v + m - 1) // m * m


NQ = 2


@functools.partial(jax.jit, static_argnames=("F", "nb", "npad", "chunked"))
def _sc_segment_sum(h_ch, src2d, dst2d, zeros_pad, F, nb, npad,
                    chunked=True):
    F_per_core = F // NCORES
    ZR = npad // NSUB
    qb = nb // NQ
    mesh = plsc.VectorSubcoreMesh(core_axis_name="c", subcore_axis_name="s")

    @functools.partial(
        pl.kernel,
        out_type=jax.ShapeDtypeStruct((F, npad, LANES), jnp.float32),
        mesh=mesh,
        scratch_types=[
            pltpu.VMEM((qb, EBLK), jnp.int32),
            pltpu.VMEM((qb, EBLK), jnp.int32),
            pltpu.VMEM((EBLK, LANES), jnp.float32),
            pltpu.VMEM((EBLK, LANES), jnp.float32),
            pltpu.VMEM_SHARED((npad, LANES), jnp.float32),
            pltpu.SemaphoreType.DMA,
            pltpu.SemaphoreType.DMA,
        ],
    )
    def segsum(h_hbm, src_hbm, dst_hbm, z_hbm, out_hbm,
               src_v, dst_v, buf0, buf1, acc, sem0, sem1):
        c = lax.axis_index("c")
        s = lax.axis_index("s")
        bufs = (buf0, buf1)
        sems = (sem0, sem1)
        for fi in range(F_per_core):
            chunk = c * F_per_core + fi
            if chunked:
                h_f = h_hbm.at[chunk]
            else:
                off = pl.multiple_of(chunk * LANES, LANES)
                h_f = h_hbm.at[:, pl.ds(off, LANES)]
            pltpu.sync_copy(z_hbm.at[pl.ds(s * ZR, ZR)],
                            acc.at[pl.ds(s * ZR, ZR)])
            plsc.subcore_barrier()

            def start(j, k, h_f=h_f):
                pltpu.make_async_copy(h_f.at[src_v.at[j]], bufs[k],
                                      sems[k]).start()

            def wait(k, h_f=h_f):
                pltpu.make_async_copy(h_f.at[src_v.at[0]], bufs[k],
                                      sems[k]).wait()

            for q in range(NQ):
                pltpu.sync_copy(src_hbm.at[pl.ds((s * NQ + q) * qb, qb)],
                                src_v)
                pltpu.sync_copy(dst_hbm.at[pl.ds((s * NQ + q) * qb, qb)],
                                dst_v)
                start(0, 0)

                def body(i, carry):
                    j0 = 2 * i
                    start(j0 + 1, 1)
                    wait(0)
                    pltpu.sync_copy(bufs[0], acc.at[dst_v.at[j0]], add=True)

                    @pl.when(j0 + 2 < qb)
                    def _():
                        start(j0 + 2, 0)

                    wait(1)
                    pltpu.sync_copy(bufs[1], acc.at[dst_v.at[j0 + 1]],
                                    add=True)
                    return carry

                lax.fori_loop(0, qb // 2, body, 0)
            plsc.subcore_barrier()
            pltpu.sync_copy(acc.at[pl.ds(s * ZR, ZR)],
                            out_hbm.at[chunk, pl.ds(s * ZR, ZR)])
            if fi + 1 < F_per_core:
                plsc.subcore_barrier()

    return segsum(h_ch, src2d, dst2d, zeros_pad)


def _h_chunk(h_ref, f):
    if len(h_ref.shape) == 3:
        return h_ref[f]
    return h_ref[:, f * LANES:(f + 1) * LANES]


def _mlp_bn_body(eps_ref, h_ref, agg_ref, w1_ref, b1_ref, w2_ref, b2_ref,
                 gamma_ref, beta_ref, n_ref, out_ref, z2s_ref, s1_ref,
                 s2_ref):
    p = pl.program_id(0)
    i = pl.program_id(1)
    F = agg_ref.shape[0]

    @pl.when(p == 0)
    def _():
        e = eps_ref[0, 0]
        acc = jnp.zeros((RB, w1_ref.shape[2]), jnp.float32)
        for f in range(F):
            zf = e * _h_chunk(h_ref, f) + agg_ref[f]
            acc += jnp.dot(zf.astype(jnp.bfloat16),
                           w1_ref[f].astype(jnp.bfloat16),
                           preferred_element_type=jnp.float32)
        a = jnp.maximum(acc + b1_ref[...], 0.0)
        z2 = jnp.dot(a.astype(jnp.bfloat16), w2_ref[...].astype(jnp.bfloat16),
                     preferred_element_type=jnp.float32) + b2_ref[...]
        z2s_ref[pl.ds(i * RB, RB), :] = z2
        ps1 = jnp.sum(z2, axis=0, keepdims=True)
        ps2 = jnp.sum(z2 * z2, axis=0, keepdims=True)

        @pl.when(i == 0)
        def _():
            s1_ref[...] = ps1
            s2_ref[...] = ps2

        @pl.when(i > 0)
        def _():
            s1_ref[...] += ps1
            s2_ref[...] += ps2

    @pl.when(p == 1)
    def _():
        inv_n = 1.0 / n_ref[0, 0]
        mean = s1_ref[...] * inv_n
        var = s2_ref[...] * inv_n - mean * mean
        a = gamma_ref[...] * lax.rsqrt(var + BN_EPS)
        b = beta_ref[...] - mean * a
        h = jnp.maximum(z2s_ref[pl.ds(i * RB, RB), :] * a + b, 0.0)
        Fo = out_ref.shape[0]
        for f in range(Fo):
            out_ref[f] = h[:, f * LANES:(f + 1) * LANES]


def _h_spec(h_ch):
    if h_ch.ndim == 3:
        return pl.BlockSpec((h_ch.shape[0], RB, LANES),
                            lambda p, i: (0, i * (1 - p), 0))
    return pl.BlockSpec((RB, h_ch.shape[1]), lambda p, i: (i * (1 - p), 0))


def _tc_mlp_bn(eps1p, h_ch, agg_ch, w1r, b1, w2, b2, gamma, beta, nf,
               n_rows):
    F = agg_ch.shape[0]
    hid = w2.shape[1]
    Fo = hid // LANES
    grid = n_rows // RB
    return pl.pallas_call(
        _mlp_bn_body,
        grid=(2, grid),
        in_specs=[
            pl.BlockSpec((1, 1), lambda p, i: (0, 0),
                         memory_space=pltpu.SMEM),
            _h_spec(h_ch),
            pl.BlockSpec((F, RB, LANES), lambda p, i: (0, i * (1 - p), 0)),
            pl.BlockSpec((F, LANES, hid), lambda p, i: (0, 0, 0)),
            pl.BlockSpec((1, hid), lambda p, i: (0, 0)),
            pl.BlockSpec((hid, hid), lambda p, i: (0, 0)),
            pl.BlockSpec((1, hid), lambda p, i: (0, 0)),
            pl.BlockSpec((1, hid), lambda p, i: (0, 0)),
            pl.BlockSpec((1, hid), lambda p, i: (0, 0)),
            pl.BlockSpec((1, 1), lambda p, i: (0, 0),
                         memory_space=pltpu.SMEM),
        ],
        out_specs=pl.BlockSpec((Fo, RB, LANES), lambda p, i: (0, i * p, 0)),
        out_shape=jax.ShapeDtypeStruct((Fo, n_rows, LANES), jnp.float32),
        scratch_shapes=[
            pltpu.VMEM((n_rows, hid), jnp.float32),
            pltpu.VMEM((1, hid), jnp.float32),
            pltpu.VMEM((1, hid), jnp.float32),
        ],
        compiler_params=pltpu.CompilerParams(
            dimension_semantics=("arbitrary", "arbitrary")),
    )(eps1p, h_ch, agg_ch, w1r, b1, w2, b2, gamma, beta, nf)


def _mlp_bn_pool_body(eps_ref, h_ref, agg_ref, w1_ref, b1_ref, w2_ref,
                      b2_ref, gamma_ref, beta_ref, n_ref, bat_ref, wc_ref,
                      bc_ref, out_ref, z2s_ref, s1_ref, s2_ref, pacc_ref):
    p = pl.program_id(0)
    i = pl.program_id(1)
    ngrid = pl.num_programs(1)
    F = agg_ref.shape[0]
    ng = pacc_ref.shape[0]

    @pl.when(p == 0)
    def _():
        e = eps_ref[0, 0]
        acc = jnp.zeros((RB, w1_ref.shape[2]), jnp.float32)
        for f in range(F):
            zf = e * _h_chunk(h_ref, f) + agg_ref[f]
            acc += jnp.dot(zf.astype(jnp.bfloat16),
                           w1_ref[f].astype(jnp.bfloat16),
                           preferred_element_type=jnp.float32)
        a = jnp.maximum(acc + b1_ref[...], 0.0)
        z2 = jnp.dot(a.astype(jnp.bfloat16), w2_ref[...].astype(jnp.bfloat16),
                     preferred_element_type=jnp.float32) + b2_ref[...]
        z2s_ref[pl.ds(i * RB, RB), :] = z2
        ps1 = jnp.sum(z2, axis=0, keepdims=True)
        ps2 = jnp.sum(z2 * z2, axis=0, keepdims=True)

        @pl.when(i == 0)
        def _():
            s1_ref[...] = ps1
            s2_ref[...] = ps2

        @pl.when(i > 0)
        def _():
            s1_ref[...] += ps1
            s2_ref[...] += ps2

    @pl.when(p == 1)
    def _():
        inv_n = 1.0 / n_ref[0, 0]
        mean = s1_ref[...] * inv_n
        var = s2_ref[...] * inv_n - mean * mean
        a = gamma_ref[...] * lax.rsqrt(var + BN_EPS)
        b = beta_ref[...] - mean * a
        h = jnp.maximum(z2s_ref[pl.ds(i * RB, RB), :] * a + b, 0.0)
        bat = bat_ref[0, 0]
        ohT = (lax.broadcasted_iota(jnp.int32, (ng, RB), 0)
               == bat[None, :]).astype(jnp.float32)
        part = jnp.dot(ohT, h, preferred_element_type=jnp.float32)

        @pl.when(i == 0)
        def _():
            pacc_ref[...] = part

        @pl.when(i > 0)
        def _():
            pacc_ref[...] += part

        @pl.when(i == ngrid - 1)
        def _():
            out_ref[...] = jnp.dot(pacc_ref[...], wc_ref[...],
                                   preferred_element_type=jnp.float32) \
                + bc_ref[...]


def _tc_mlp_bn_pool(eps1p, h_ch, agg_ch, w1r, b1, w2, b2, gamma, beta, nf,
                    bat3d, wc, bc, n_rows, num_graphs):
    F = agg_ch.shape[0]
    hid = w2.shape[1]
    ncls = wc.shape[1]
    grid = n_rows // RB
    return pl.pallas_call(
        _mlp_bn_pool_body,
        grid=(2, grid),
        in_specs=[
            pl.BlockSpec((1, 1), lambda p, i: (0, 0),
                         memory_space=pltpu.SMEM),
            _h_spec(h_ch),
            pl.BlockSpec((F, RB, LANES), lambda p, i: (0, i * (1 - p), 0)),
            pl.BlockSpec((F, LANES, hid), lambda p, i: (0, 0, 0)),
            pl.BlockSpec((1, hid), lambda p, i: (0, 0)),
            pl.BlockSpec((hid, hid), lambda p, i: (0, 0)),
            pl.BlockSpec((1, hid), lambda p, i: (0, 0)),
            pl.BlockSpec((1, hid), lambda p, i: (0, 0)),
            pl.BlockSpec((1, hid), lambda p, i: (0, 0)),
            pl.BlockSpec((1, 1), lambda p, i: (0, 0),
                         memory_space=pltpu.SMEM),
            pl.BlockSpec((1, 1, RB), lambda p, i: (i, 0, 0)),
            pl.BlockSpec((hid, ncls), lambda p, i: (0, 0)),
            pl.BlockSpec((1, ncls), lambda p, i: (0, 0)),
        ],
        out_specs=pl.BlockSpec((num_graphs, ncls), lambda p, i: (0, 0)),
        out_shape=jax.ShapeDtypeStruct((num_graphs, ncls), jnp.float32),
        scratch_shapes=[
            pltpu.VMEM((n_rows, hid), jnp.float32),
            pltpu.VMEM((1, hid), jnp.float32),
            pltpu.VMEM((1, hid), jnp.float32),
            pltpu.VMEM((num_graphs, hid), jnp.float32),
        ],
        compiler_params=pltpu.CompilerParams(
            dimension_semantics=("arbitrary", "arbitrary")),
    )(eps1p, h_ch, agg_ch, w1r, b1, w2, b2, gamma, beta, nf, bat3d, wc, bc)


def kernel(x, edge_index, batch, params):
    n, in_dim = x.shape
    e_edges = edge_index.shape[1]
    num_graphs = 64
    ncls = params['Wc'].shape[1]

    npad = _round_up(n + NSUB, NSUB * 8)
    nb = _round_up((e_edges + NSUB * EBLK - 1) // (NSUB * EBLK), 8 * NQ)
    e_pad = NSUB * nb * EBLK - e_edges

    src = edge_index[0]
    dst = edge_index[1]
    pad_ar = jnp.arange(e_pad, dtype=jnp.int32)
    src_p = jnp.concatenate([src, (pad_ar * 1009) % n]).reshape(NSUB * nb, EBLK)
    dst_p = jnp.concatenate([dst, n + (pad_ar % NSUB)]).reshape(NSUB * nb, EBLK)
    zeros_pad = jnp.zeros((npad, LANES), jnp.float32)

    h_ch = x

    nf = jnp.full((1, 1), float(n), jnp.float32)
    bat3d = batch.reshape(n // RB, 1, RB)
    n_layers = len(params['layers'])
    for li, lp in enumerate(params['layers']):
        F = (h_ch.shape[0] if h_ch.ndim == 3 else h_ch.shape[1] // LANES)
        agg_ch = _sc_segment_sum(h_ch, src_p, dst_p, zeros_pad,
                                 F=F, nb=nb, npad=npad,
                                 chunked=(h_ch.ndim == 3))
        eps1p = (1.0 + lp['eps']).reshape(1, 1).astype(jnp.float32)
        hid = lp['W2'].shape[1]
        w1r = lp['W1'].reshape(F, LANES, hid)
        args = (eps1p, h_ch, agg_ch, w1r, lp['b1'].reshape(1, hid),
                lp['W2'], lp['b2'].reshape(1, hid),
                lp['gamma'].reshape(1, hid), lp['beta'].reshape(1, hid), nf)
        if li + 1 < n_layers:
            h_ch = _tc_mlp_bn(*args, n)
        else:
            return _tc_mlp_bn_pool(*args, bat3d, params['Wc'],
                                   params['bc'].reshape(1, ncls), n,
                                   num_graphs)


# speedup vs baseline: 1.1603x; 1.0601x over previous
import functools

import jax
import jax.numpy as jnp
from jax import lax
from jax.experimental import pallas as pl
from jax.experimental.pallas import tpu as pltpu
from jax.experimental.pallas import tpu_sc as plsc

BN_EPS = 1e-5
LANES = 128
EBLK = 128
NSUB = 16
NCORES = 2
RB = 1000


def _round_up(v, m):
    return (v + m - 1) // m * m


NQ = 2


@functools.partial(jax.jit, static_argnames=("F", "nb", "npad", "chunked"))
def _sc_segment_sum(h_ch, src2d, dst2d, zeros_pad, F, nb, npad,
                    chunked=True):
    F_per_core = F // NCORES
    ZR = npad // NSUB
    qb = nb // NQ
    mesh = plsc.VectorSubcoreMesh(core_axis_name="c", subcore_axis_name="s")

    @functools.partial(
        pl.kernel,
        out_type=jax.ShapeDtypeStruct((F, npad, LANES), jnp.float32),
        mesh=mesh,
        scratch_types=[
            pltpu.VMEM((qb, EBLK), jnp.int32),
            pltpu.VMEM((qb, EBLK), jnp.int32),
            pltpu.VMEM((EBLK, LANES), jnp.float32),
            pltpu.VMEM((EBLK, LANES), jnp.float32),
            pltpu.VMEM_SHARED((npad, LANES), jnp.float32),
            pltpu.SemaphoreType.DMA,
            pltpu.SemaphoreType.DMA,
        ],
    )
    def segsum(h_hbm, src_hbm, dst_hbm, z_hbm, out_hbm,
               src_v, dst_v, buf0, buf1, acc, sem0, sem1):
        c = lax.axis_index("c")
        s = lax.axis_index("s")
        bufs = (buf0, buf1)
        sems = (sem0, sem1)
        for fi in range(F_per_core):
            chunk = c * F_per_core + fi
            if chunked:
                h_f = h_hbm.at[chunk]
            else:
                off = pl.multiple_of(chunk * LANES, LANES)
                h_f = h_hbm.at[:, pl.ds(off, LANES)]
            pltpu.sync_copy(z_hbm.at[pl.ds(s * ZR, ZR)],
                            acc.at[pl.ds(s * ZR, ZR)])
            plsc.subcore_barrier()

            def start(j, k, h_f=h_f):
                pltpu.make_async_copy(h_f.at[src_v.at[j]], bufs[k],
                                      sems[k]).start()

            def wait(k, h_f=h_f):
                pltpu.make_async_copy(h_f.at[src_v.at[0]], bufs[k],
                                      sems[k]).wait()

            for q in range(NQ):
                pltpu.sync_copy(src_hbm.at[pl.ds((s * NQ + q) * qb, qb)],
                                src_v)
                pltpu.sync_copy(dst_hbm.at[pl.ds((s * NQ + q) * qb, qb)],
                                dst_v)
                start(0, 0)

                def body(i, carry):
                    j0 = 2 * i
                    start(j0 + 1, 1)
                    wait(0)
                    pltpu.sync_copy(bufs[0], acc.at[dst_v.at[j0]], add=True)

                    @pl.when(j0 + 2 < qb)
                    def _():
                        start(j0 + 2, 0)

                    wait(1)
                    pltpu.sync_copy(bufs[1], acc.at[dst_v.at[j0 + 1]],
                                    add=True)
                    return carry

                lax.fori_loop(0, qb // 2, body, 0)
            plsc.subcore_barrier()
            pltpu.sync_copy(acc.at[pl.ds(s * ZR, ZR)],
                            out_hbm.at[chunk, pl.ds(s * ZR, ZR)])
            if fi + 1 < F_per_core:
                plsc.subcore_barrier()

    return segsum(h_ch, src2d, dst2d, zeros_pad)


def _h_chunk(h_ref, f):
    if len(h_ref.shape) == 3:
        return h_ref[f]
    return h_ref[:, f * LANES:(f + 1) * LANES]


def _mlp_bn_body(eps_ref, h_ref, agg_ref, w1_ref, b1_ref, w2_ref, b2_ref,
                 gamma_ref, beta_ref, n_ref, out_ref, z2s_ref, s1_ref,
                 s2_ref):
    p = pl.program_id(0)
    i = pl.program_id(1)
    F = agg_ref.shape[0]

    @pl.when(p == 0)
    def _():
        e = eps_ref[0, 0]
        acc = jnp.zeros((RB, w1_ref.shape[2]), jnp.float32)
        for f in range(F):
            zf = e * _h_chunk(h_ref, f) + agg_ref[f]
            acc += jnp.dot(zf.astype(jnp.bfloat16),
                           w1_ref[f].astype(jnp.bfloat16),
                           preferred_element_type=jnp.float32)
        a = jnp.maximum(acc + b1_ref[...], 0.0)
        z2 = jnp.dot(a.astype(jnp.bfloat16), w2_ref[...].astype(jnp.bfloat16),
                     preferred_element_type=jnp.float32) + b2_ref[...]
        z2s_ref[pl.ds(i * RB, RB), :] = z2
        ps1 = jnp.sum(z2, axis=0, keepdims=True)
        ps2 = jnp.sum(z2 * z2, axis=0, keepdims=True)

        @pl.when(i == 0)
        def _():
            s1_ref[...] = ps1
            s2_ref[...] = ps2

        @pl.when(i > 0)
        def _():
            s1_ref[...] += ps1
            s2_ref[...] += ps2

    @pl.when(p == 1)
    def _():
        inv_n = 1.0 / n_ref[0, 0]
        mean = s1_ref[...] * inv_n
        var = s2_ref[...] * inv_n - mean * mean
        a = gamma_ref[...] * lax.rsqrt(var + BN_EPS)
        b = beta_ref[...] - mean * a
        h = jnp.maximum(z2s_ref[pl.ds(i * RB, RB), :] * a + b, 0.0)
        Fo = out_ref.shape[0]
        for f in range(Fo):
            out_ref[f] = h[:, f * LANES:(f + 1) * LANES]


def _h_spec(h_ch):
    if h_ch.ndim == 3:
        return pl.BlockSpec((h_ch.shape[0], RB, LANES),
                            lambda p, i: (0, i * (1 - p), 0))
    return pl.BlockSpec((RB, h_ch.shape[1]), lambda p, i: (i * (1 - p), 0))


def _tc_mlp_bn(eps1p, h_ch, agg_ch, w1r, b1, w2, b2, gamma, beta, nf,
               n_rows):
    F = agg_ch.shape[0]
    hid = w2.shape[1]
    Fo = hid // LANES
    grid = n_rows // RB
    return pl.pallas_call(
        _mlp_bn_body,
        grid=(2, grid),
        in_specs=[
            pl.BlockSpec((1, 1), lambda p, i: (0, 0),
                         memory_space=pltpu.SMEM),
            _h_spec(h_ch),
            pl.BlockSpec((F, RB, LANES), lambda p, i: (0, i * (1 - p), 0)),
            pl.BlockSpec((F, LANES, hid), lambda p, i: (0, 0, 0)),
            pl.BlockSpec((1, hid), lambda p, i: (0, 0)),
            pl.BlockSpec((hid, hid), lambda p, i: (0, 0)),
            pl.BlockSpec((1, hid), lambda p, i: (0, 0)),
            pl.BlockSpec((1, hid), lambda p, i: (0, 0)),
            pl.BlockSpec((1, hid), lambda p, i: (0, 0)),
            pl.BlockSpec((1, 1), lambda p, i: (0, 0),
                         memory_space=pltpu.SMEM),
        ],
        out_specs=pl.BlockSpec((Fo, RB, LANES), lambda p, i: (0, i * p, 0)),
        out_shape=jax.ShapeDtypeStruct((Fo, n_rows, LANES), jnp.float32),
        scratch_shapes=[
            pltpu.VMEM((n_rows, hid), jnp.float32),
            pltpu.VMEM((1, hid), jnp.float32),
            pltpu.VMEM((1, hid), jnp.float32),
        ],
        compiler_params=pltpu.CompilerParams(
            dimension_semantics=("arbitrary", "arbitrary")),
    )(eps1p, h_ch, agg_ch, w1r, b1, w2, b2, gamma, beta, nf)


def _mlp_bn_pool_body(eps_ref, h_ref, agg_ref, w1_ref, b1_ref, w2_ref,
                      b2_ref, gamma_ref, beta_ref, n_ref, bat_ref, wc_ref,
                      bc_ref, out_ref, z2s_ref, s1_ref, s2_ref, pacc_ref):
    p = pl.program_id(0)
    i = pl.program_id(1)
    ngrid = pl.num_programs(1)
    F = agg_ref.shape[0]
    ng = pacc_ref.shape[0]

    @pl.when(p == 0)
    def _():
        e = eps_ref[0, 0]
        acc = jnp.zeros((RB, w1_ref.shape[2]), jnp.float32)
        for f in range(F):
            zf = e * _h_chunk(h_ref, f) + agg_ref[f]
            acc += jnp.dot(zf.astype(jnp.bfloat16),
                           w1_ref[f].astype(jnp.bfloat16),
                           preferred_element_type=jnp.float32)
        a = jnp.maximum(acc + b1_ref[...], 0.0)
        z2 = jnp.dot(a.astype(jnp.bfloat16), w2_ref[...].astype(jnp.bfloat16),
                     preferred_element_type=jnp.float32) + b2_ref[...]
        z2s_ref[pl.ds(i * RB, RB), :] = z2
        ps1 = jnp.sum(z2, axis=0, keepdims=True)
        ps2 = jnp.sum(z2 * z2, axis=0, keepdims=True)

        @pl.when(i == 0)
        def _():
            s1_ref[...] = ps1
            s2_ref[...] = ps2

        @pl.when(i > 0)
        def _():
            s1_ref[...] += ps1
            s2_ref[...] += ps2

    @pl.when(p == 1)
    def _():
        inv_n = 1.0 / n_ref[0, 0]
        mean = s1_ref[...] * inv_n
        var = s2_ref[...] * inv_n - mean * mean
        a = gamma_ref[...] * lax.rsqrt(var + BN_EPS)
        b = beta_ref[...] - mean * a
        h = jnp.maximum(z2s_ref[pl.ds(i * RB, RB), :] * a + b, 0.0)
        bat = bat_ref[0, 0]
        ohT = (lax.broadcasted_iota(jnp.int32, (ng, RB), 0)
               == bat[None, :]).astype(jnp.float32)
        part = jnp.dot(ohT, h, preferred_element_type=jnp.float32)

        @pl.when(i == 0)
        def _():
            pacc_ref[...] = part

        @pl.when(i > 0)
        def _():
            pacc_ref[...] += part

        @pl.when(i == ngrid - 1)
        def _():
            out_ref[...] = jnp.dot(pacc_ref[...], wc_ref[...],
                                   preferred_element_type=jnp.float32) \
                + bc_ref[...]


def _tc_mlp_bn_pool(eps1p, h_ch, agg_ch, w1r, b1, w2, b2, gamma, beta, nf,
                    bat3d, wc, bc, n_rows, num_graphs):
    F = agg_ch.shape[0]
    hid = w2.shape[1]
    ncls = wc.shape[1]
    grid = n_rows // RB
    return pl.pallas_call(
        _mlp_bn_pool_body,
        grid=(2, grid),
        in_specs=[
            pl.BlockSpec((1, 1), lambda p, i: (0, 0),
                         memory_space=pltpu.SMEM),
            _h_spec(h_ch),
            pl.BlockSpec((F, RB, LANES), lambda p, i: (0, i * (1 - p), 0)),
            pl.BlockSpec((F, LANES, hid), lambda p, i: (0, 0, 0)),
            pl.BlockSpec((1, hid), lambda p, i: (0, 0)),
            pl.BlockSpec((hid, hid), lambda p, i: (0, 0)),
            pl.BlockSpec((1, hid), lambda p, i: (0, 0)),
            pl.BlockSpec((1, hid), lambda p, i: (0, 0)),
            pl.BlockSpec((1, hid), lambda p, i: (0, 0)),
            pl.BlockSpec((1, 1), lambda p, i: (0, 0),
                         memory_space=pltpu.SMEM),
            pl.BlockSpec((1, 1, RB), lambda p, i: (i, 0, 0)),
            pl.BlockSpec((hid, ncls), lambda p, i: (0, 0)),
            pl.BlockSpec((1, ncls), lambda p, i: (0, 0)),
        ],
        out_specs=pl.BlockSpec((num_graphs, ncls), lambda p, i: (0, 0)),
        out_shape=jax.ShapeDtypeStruct((num_graphs, ncls), jnp.float32),
        scratch_shapes=[
            pltpu.VMEM((n_rows, hid), jnp.float32),
            pltpu.VMEM((1, hid), jnp.float32),
            pltpu.VMEM((1, hid), jnp.float32),
            pltpu.VMEM((num_graphs, hid), jnp.float32),
        ],
        compiler_params=pltpu.CompilerParams(
            dimension_semantics=("arbitrary", "arbitrary")),
    )(eps1p, h_ch, agg_ch, w1r, b1, w2, b2, gamma, beta, nf, bat3d, wc, bc)


def kernel(x, edge_index, batch, params):
    n, in_dim = x.shape
    e_edges = edge_index.shape[1]
    num_graphs = 64
    ncls = params['Wc'].shape[1]

    npad = _round_up(n + NSUB, NSUB * 8)
    nb = _round_up((e_edges + NSUB * EBLK - 1) // (NSUB * EBLK), 8 * NQ)
    e_pad = NSUB * nb * EBLK - e_edges

    src = edge_index[0]
    dst = edge_index[1]
    pad_ar = jnp.arange(e_pad, dtype=jnp.int32)
    src_p = jnp.concatenate([src, (pad_ar * 1009) % n]).reshape(NSUB * nb, EBLK)
    dst_p = jnp.concatenate([dst, n + (pad_ar % NSUB)]).reshape(NSUB * nb, EBLK)
    zeros_pad = jnp.zeros((npad, LANES), jnp.float32)

    h_ch = x

    nf = jnp.full((1, 1), float(n), jnp.float32)
    bat3d = batch.reshape(n // RB, 1, RB)
    n_layers = len(params['layers'])
    for li, lp in enumerate(params['layers']):
        F = (h_ch.shape[0] if h_ch.ndim == 3 else h_ch.shape[1] // LANES)
        agg_ch = _sc_segment_sum(h_ch, src_p, dst_p, zeros_pad,
                                 F=F, nb=nb, npad=npad,
                                 chunked=(h_ch.ndim == 3))
        eps1p = (1.0 + lp['eps']).reshape(1, 1).astype(jnp.float32)
        hid = lp['W2'].shape[1]
        w1r = lp['W1'].reshape(F, LANES, hid)
        args = (eps1p, h_ch, agg_ch, w1r, lp['b1'].reshape(1, hid),
                lp['W2'], lp['b2'].reshape(1, hid),
                lp['gamma'].reshape(1, hid), lp['beta'].reshape(1, hid), nf)
        if li + 1 < n_layers:
            h_ch = _tc_mlp_bn(*args, n)
        else:
            return _tc_mlp_bn_pool(*args, bat3d, params['Wc'],
                                   params['bc'].reshape(1, ncls), n,
                                   num_graphs)


# speedup vs baseline: 1.1729x; 1.0108x over previous
import functools

import jax
import jax.numpy as jnp
from jax import lax
from jax.experimental import pallas as pl
from jax.experimental.pallas import tpu as pltpu
from jax.experimental.pallas import tpu_sc as plsc

BN_EPS = 1e-5
LANES = 128
EBLK = 128
NSUB = 16
NCORES = 2
RB = 2000


def _round_up(v, m):
    return (v + m - 1) // m * m


NQ = 2


@functools.partial(jax.jit, static_argnames=("F", "nb", "npad", "chunked"))
def _sc_segment_sum(h_ch, src2d, dst2d, zeros_pad, F, nb, npad,
                    chunked=True):
    F_per_core = F // NCORES
    ZR = npad // NSUB
    qb = nb // NQ
    mesh = plsc.VectorSubcoreMesh(core_axis_name="c", subcore_axis_name="s")

    @functools.partial(
        pl.kernel,
        out_type=jax.ShapeDtypeStruct((F, npad, LANES), jnp.float32),
        mesh=mesh,
        scratch_types=[
            pltpu.VMEM((qb, EBLK), jnp.int32),
            pltpu.VMEM((qb, EBLK), jnp.int32),
            pltpu.VMEM((EBLK, LANES), jnp.float32),
            pltpu.VMEM((EBLK, LANES), jnp.float32),
            pltpu.VMEM_SHARED((npad, LANES), jnp.float32),
            pltpu.SemaphoreType.DMA,
            pltpu.SemaphoreType.DMA,
        ],
    )
    def segsum(h_hbm, src_hbm, dst_hbm, z_hbm, out_hbm,
               src_v, dst_v, buf0, buf1, acc, sem0, sem1):
        c = lax.axis_index("c")
        s = lax.axis_index("s")
        bufs = (buf0, buf1)
        sems = (sem0, sem1)
        for fi in range(F_per_core):
            chunk = c * F_per_core + fi
            if chunked:
                h_f = h_hbm.at[chunk]
            else:
                off = pl.multiple_of(chunk * LANES, LANES)
                h_f = h_hbm.at[:, pl.ds(off, LANES)]
            pltpu.sync_copy(z_hbm.at[pl.ds(s * ZR, ZR)],
                            acc.at[pl.ds(s * ZR, ZR)])
            plsc.subcore_barrier()

            def start(j, k, h_f=h_f):
                pltpu.make_async_copy(h_f.at[src_v.at[j]], bufs[k],
                                      sems[k]).start()

            def wait(k, h_f=h_f):
                pltpu.make_async_copy(h_f.at[src_v.at[0]], bufs[k],
                                      sems[k]).wait()

            for q in range(NQ):
                pltpu.sync_copy(src_hbm.at[pl.ds((s * NQ + q) * qb, qb)],
                                src_v)
                pltpu.sync_copy(dst_hbm.at[pl.ds((s * NQ + q) * qb, qb)],
                                dst_v)
                start(0, 0)

                def body(i, carry):
                    j0 = 2 * i
                    start(j0 + 1, 1)
                    wait(0)
                    pltpu.sync_copy(bufs[0], acc.at[dst_v.at[j0]], add=True)

                    @pl.when(j0 + 2 < qb)
                    def _():
                        start(j0 + 2, 0)

                    wait(1)
                    pltpu.sync_copy(bufs[1], acc.at[dst_v.at[j0 + 1]],
                                    add=True)
                    return carry

                lax.fori_loop(0, qb // 2, body, 0)
            plsc.subcore_barrier()
            pltpu.sync_copy(acc.at[pl.ds(s * ZR, ZR)],
                            out_hbm.at[chunk, pl.ds(s * ZR, ZR)])
            if fi + 1 < F_per_core:
                plsc.subcore_barrier()

    return segsum(h_ch, src2d, dst2d, zeros_pad)


def _h_chunk(h_ref, f):
    if len(h_ref.shape) == 3:
        return h_ref[f]
    return h_ref[:, f * LANES:(f + 1) * LANES]


def _mlp_bn_body(eps_ref, h_ref, agg_ref, w1_ref, b1_ref, w2_ref, b2_ref,
                 gamma_ref, beta_ref, n_ref, out_ref, z2s_ref, s1_ref,
                 s2_ref):
    p = pl.program_id(0)
    i = pl.program_id(1)
    F = agg_ref.shape[0]

    @pl.when(p == 0)
    def _():
        e = eps_ref[0, 0]
        acc = jnp.zeros((RB, w1_ref.shape[2]), jnp.float32)
        for f in range(F):
            zf = e * _h_chunk(h_ref, f) + agg_ref[f]
            acc += jnp.dot(zf.astype(jnp.bfloat16),
                           w1_ref[f].astype(jnp.bfloat16),
                           preferred_element_type=jnp.float32)
        a = jnp.maximum(acc + b1_ref[...], 0.0)
        z2 = jnp.dot(a.astype(jnp.bfloat16), w2_ref[...].astype(jnp.bfloat16),
                     preferred_element_type=jnp.float32) + b2_ref[...]
        z2s_ref[pl.ds(i * RB, RB), :] = z2
        ps1 = jnp.sum(z2, axis=0, keepdims=True)
        ps2 = jnp.sum(z2 * z2, axis=0, keepdims=True)

        @pl.when(i == 0)
        def _():
            s1_ref[...] = ps1
            s2_ref[...] = ps2

        @pl.when(i > 0)
        def _():
            s1_ref[...] += ps1
            s2_ref[...] += ps2

    @pl.when(p == 1)
    def _():
        inv_n = 1.0 / n_ref[0, 0]
        mean = s1_ref[...] * inv_n
        var = s2_ref[...] * inv_n - mean * mean
        a = gamma_ref[...] * lax.rsqrt(var + BN_EPS)
        b = beta_ref[...] - mean * a
        h = jnp.maximum(z2s_ref[pl.ds(i * RB, RB), :] * a + b, 0.0)
        Fo = out_ref.shape[0]
        for f in range(Fo):
            out_ref[f] = h[:, f * LANES:(f + 1) * LANES]


def _h_spec(h_ch):
    if h_ch.ndim == 3:
        return pl.BlockSpec((h_ch.shape[0], RB, LANES),
                            lambda p, i: (0, i * (1 - p), 0))
    return pl.BlockSpec((RB, h_ch.shape[1]), lambda p, i: (i * (1 - p), 0))


def _tc_mlp_bn(eps1p, h_ch, agg_ch, w1r, b1, w2, b2, gamma, beta, nf,
               n_rows):
    F = agg_ch.shape[0]
    hid = w2.shape[1]
    Fo = hid // LANES
    grid = n_rows // RB
    return pl.pallas_call(
        _mlp_bn_body,
        grid=(2, grid),
        in_specs=[
            pl.BlockSpec((1, 1), lambda p, i: (0, 0),
                         memory_space=pltpu.SMEM),
            _h_spec(h_ch),
            pl.BlockSpec((F, RB, LANES), lambda p, i: (0, i * (1 - p), 0)),
            pl.BlockSpec((F, LANES, hid), lambda p, i: (0, 0, 0)),
            pl.BlockSpec((1, hid), lambda p, i: (0, 0)),
            pl.BlockSpec((hid, hid), lambda p, i: (0, 0)),
            pl.BlockSpec((1, hid), lambda p, i: (0, 0)),
            pl.BlockSpec((1, hid), lambda p, i: (0, 0)),
            pl.BlockSpec((1, hid), lambda p, i: (0, 0)),
            pl.BlockSpec((1, 1), lambda p, i: (0, 0),
                         memory_space=pltpu.SMEM),
        ],
        out_specs=pl.BlockSpec((Fo, RB, LANES), lambda p, i: (0, i * p, 0)),
        out_shape=jax.ShapeDtypeStruct((Fo, n_rows, LANES), jnp.float32),
        scratch_shapes=[
            pltpu.VMEM((n_rows, hid), jnp.float32),
            pltpu.VMEM((1, hid), jnp.float32),
            pltpu.VMEM((1, hid), jnp.float32),
        ],
        compiler_params=pltpu.CompilerParams(
            dimension_semantics=("arbitrary", "arbitrary")),
    )(eps1p, h_ch, agg_ch, w1r, b1, w2, b2, gamma, beta, nf)


def _mlp_bn_pool_body(eps_ref, h_ref, agg_ref, w1_ref, b1_ref, w2_ref,
                      b2_ref, gamma_ref, beta_ref, n_ref, bat_ref, wc_ref,
                      bc_ref, out_ref, z2s_ref, s1_ref, s2_ref, pacc_ref):
    p = pl.program_id(0)
    i = pl.program_id(1)
    ngrid = pl.num_programs(1)
    F = agg_ref.shape[0]
    ng = pacc_ref.shape[0]

    @pl.when(p == 0)
    def _():
        e = eps_ref[0, 0]
        acc = jnp.zeros((RB, w1_ref.shape[2]), jnp.float32)
        for f in range(F):
            zf = e * _h_chunk(h_ref, f) + agg_ref[f]
            acc += jnp.dot(zf.astype(jnp.bfloat16),
                           w1_ref[f].astype(jnp.bfloat16),
                           preferred_element_type=jnp.float32)
        a = jnp.maximum(acc + b1_ref[...], 0.0)
        z2 = jnp.dot(a.astype(jnp.bfloat16), w2_ref[...].astype(jnp.bfloat16),
                     preferred_element_type=jnp.float32) + b2_ref[...]
        z2s_ref[pl.ds(i * RB, RB), :] = z2
        ps1 = jnp.sum(z2, axis=0, keepdims=True)
        ps2 = jnp.sum(z2 * z2, axis=0, keepdims=True)

        @pl.when(i == 0)
        def _():
            s1_ref[...] = ps1
            s2_ref[...] = ps2

        @pl.when(i > 0)
        def _():
            s1_ref[...] += ps1
            s2_ref[...] += ps2

    @pl.when(p == 1)
    def _():
        inv_n = 1.0 / n_ref[0, 0]
        mean = s1_ref[...] * inv_n
        var = s2_ref[...] * inv_n - mean * mean
        a = gamma_ref[...] * lax.rsqrt(var + BN_EPS)
        b = beta_ref[...] - mean * a
        h = jnp.maximum(z2s_ref[pl.ds(i * RB, RB), :] * a + b, 0.0)
        bat = bat_ref[0, 0]
        ohT = (lax.broadcasted_iota(jnp.int32, (ng, RB), 0)
               == bat[None, :]).astype(jnp.float32)
        part = jnp.dot(ohT, h, preferred_element_type=jnp.float32)

        @pl.when(i == 0)
        def _():
            pacc_ref[...] = part

        @pl.when(i > 0)
        def _():
            pacc_ref[...] += part

        @pl.when(i == ngrid - 1)
        def _():
            out_ref[...] = jnp.dot(pacc_ref[...], wc_ref[...],
                                   preferred_element_type=jnp.float32) \
                + bc_ref[...]


def _tc_mlp_bn_pool(eps1p, h_ch, agg_ch, w1r, b1, w2, b2, gamma, beta, nf,
                    bat3d, wc, bc, n_rows, num_graphs):
    F = agg_ch.shape[0]
    hid = w2.shape[1]
    ncls = wc.shape[1]
    grid = n_rows // RB
    return pl.pallas_call(
        _mlp_bn_pool_body,
        grid=(2, grid),
        in_specs=[
            pl.BlockSpec((1, 1), lambda p, i: (0, 0),
                         memory_space=pltpu.SMEM),
            _h_spec(h_ch),
            pl.BlockSpec((F, RB, LANES), lambda p, i: (0, i * (1 - p), 0)),
            pl.BlockSpec((F, LANES, hid), lambda p, i: (0, 0, 0)),
            pl.BlockSpec((1, hid), lambda p, i: (0, 0)),
            pl.BlockSpec((hid, hid), lambda p, i: (0, 0)),
            pl.BlockSpec((1, hid), lambda p, i: (0, 0)),
            pl.BlockSpec((1, hid), lambda p, i: (0, 0)),
            pl.BlockSpec((1, hid), lambda p, i: (0, 0)),
            pl.BlockSpec((1, 1), lambda p, i: (0, 0),
                         memory_space=pltpu.SMEM),
            pl.BlockSpec((1, 1, RB), lambda p, i: (i, 0, 0)),
            pl.BlockSpec((hid, ncls), lambda p, i: (0, 0)),
            pl.BlockSpec((1, ncls), lambda p, i: (0, 0)),
        ],
        out_specs=pl.BlockSpec((num_graphs, ncls), lambda p, i: (0, 0)),
        out_shape=jax.ShapeDtypeStruct((num_graphs, ncls), jnp.float32),
        scratch_shapes=[
            pltpu.VMEM((n_rows, hid), jnp.float32),
            pltpu.VMEM((1, hid), jnp.float32),
            pltpu.VMEM((1, hid), jnp.float32),
            pltpu.VMEM((num_graphs, hid), jnp.float32),
        ],
        compiler_params=pltpu.CompilerParams(
            dimension_semantics=("arbitrary", "arbitrary")),
    )(eps1p, h_ch, agg_ch, w1r, b1, w2, b2, gamma, beta, nf, bat3d, wc, bc)


def kernel(x, edge_index, batch, params):
    n, in_dim = x.shape
    e_edges = edge_index.shape[1]
    num_graphs = 64
    ncls = params['Wc'].shape[1]

    npad = _round_up(n + NSUB, NSUB * 8)
    nb = _round_up((e_edges + NSUB * EBLK - 1) // (NSUB * EBLK), 8 * NQ)
    e_pad = NSUB * nb * EBLK - e_edges

    src = edge_index[0]
    dst = edge_index[1]
    pad_ar = jnp.arange(e_pad, dtype=jnp.int32)
    src_p = jnp.concatenate([src, (pad_ar * 1009) % n]).reshape(NSUB * nb, EBLK)
    dst_p = jnp.concatenate([dst, n + (pad_ar % NSUB)]).reshape(NSUB * nb, EBLK)
    zeros_pad = jnp.zeros((npad, LANES), jnp.float32)

    h_ch = x

    nf = jnp.full((1, 1), float(n), jnp.float32)
    bat3d = batch.reshape(n // RB, 1, RB)
    n_layers = len(params['layers'])
    for li, lp in enumerate(params['layers']):
        F = (h_ch.shape[0] if h_ch.ndim == 3 else h_ch.shape[1] // LANES)
        agg_ch = _sc_segment_sum(h_ch, src_p, dst_p, zeros_pad,
                                 F=F, nb=nb, npad=npad,
                                 chunked=(h_ch.ndim == 3))
        eps1p = (1.0 + lp['eps']).reshape(1, 1).astype(jnp.float32)
        hid = lp['W2'].shape[1]
        w1r = lp['W1'].reshape(F, LANES, hid)
        args = (eps1p, h_ch, agg_ch, w1r, lp['b1'].reshape(1, hid),
                lp['W2'], lp['b2'].reshape(1, hid),
                lp['gamma'].reshape(1, hid), lp['beta'].reshape(1, hid), nf)
        if li + 1 < n_layers:
            h_ch = _tc_mlp_bn(*args, n)
        else:
            return _tc_mlp_bn_pool(*args, bat3d, params['Wc'],
                                   params['bc'].reshape(1, ncls), n,
                                   num_graphs)
